# baseline (speedup 1.0000x reference)
import jax
import jax.numpy as jnp
from jax.experimental import pallas as pl


def _add_kernel(patch_ref, pos_ref, out_ref):
    out_ref[...] = patch_ref[...] + pos_ref[...]


_BB = 8
_NS = 2


def kernel(patch, pos_table):
    B, N, D = patch.shape
    NB = N // _NS
    return pl.pallas_call(
        _add_kernel,
        grid=(B // _BB, _NS),
        in_specs=[
            pl.BlockSpec((_BB, NB, D), lambda b, n: (b, n, 0)),
            pl.BlockSpec((NB, D), lambda b, n: (n, 0)),
        ],
        out_specs=pl.BlockSpec((_BB, NB, D), lambda b, n: (b, n, 0)),
        out_shape=jax.ShapeDtypeStruct((B, N, D), patch.dtype),
    )(patch, pos_table)


# speedup vs baseline: 1.0788x; 1.0788x over previous
import jax
import jax.numpy as jnp
from jax.experimental import pallas as pl


def _add_kernel(patch_ref, pos_ref, out_ref):
    out_ref[...] = patch_ref[...] + pos_ref[...]


_BB = 8
_NS = 1


def kernel(patch, pos_table):
    B, N, D = patch.shape
    NB = N // _NS
    return pl.pallas_call(
        _add_kernel,
        grid=(B // _BB, _NS),
        in_specs=[
            pl.BlockSpec((_BB, NB, D), lambda b, n: (b, n, 0)),
            pl.BlockSpec((NB, D), lambda b, n: (n, 0)),
        ],
        out_specs=pl.BlockSpec((_BB, NB, D), lambda b, n: (b, n, 0)),
        out_shape=jax.ShapeDtypeStruct((B, N, D), patch.dtype),
    )(patch, pos_table)
